# Initial kernel scaffold; baseline (speedup 1.0000x reference)
import jax
import jax.numpy as jnp
from jax.experimental import pallas as pl


def kernel(x, edge_index, W1, b1, W2, b2, Wd, bd):
    raise NotImplementedError("write your pallas kernel here")


# speedup vs baseline: 12.3224x; 12.3224x over previous
import jax
import jax.numpy as jnp
from jax import lax
from jax.experimental import pallas as pl
from jax.experimental.pallas import tpu as pltpu
from jax.experimental.pallas import tpu_sc as plsc

N = 10000
E = 320000
NC = 2
NS = 16
NW = NC * NS
BLK = 128
EPAD = ((E + NW * BLK - 1) // (NW * BLK)) * (NW * BLK)
BPT = EPAD // (NW * BLK)
RACC = 10112
RPS = RACC // NS
D = 128

_mesh = plsc.VectorSubcoreMesh(core_axis_name="c", subcore_axis_name="s")


def _sc_hist_body(dst_hbm, zeros_hbm, ones_hbm, out_hbm, dst_v, ones_v,
                  acc_sh, sem):
    c = lax.axis_index("c")
    s = lax.axis_index("s")
    wid = s * NC + c
    pltpu.sync_copy(zeros_hbm.at[pl.ds(s * RPS, RPS)],
                    acc_sh.at[pl.ds(s * RPS, RPS)])
    pltpu.sync_copy(ones_hbm, ones_v)
    pltpu.sync_copy(dst_hbm.at[wid], dst_v)
    plsc.subcore_barrier()

    def body(j, carry):
        pltpu.sync_copy(ones_v, acc_sh.at[dst_v.at[j]], add=True)
        return carry

    lax.fori_loop(0, BPT, body, 0)
    plsc.subcore_barrier()
    pltpu.sync_copy(acc_sh.at[pl.ds(s * RPS, RPS)],
                    out_hbm.at[c, pl.ds(s * RPS, RPS)])


def _sc_scatter_body(table_hbm, src_hbm, dst_hbm, zeros_hbm, out_hbm,
                     src_v, dst_v, rows_v, acc_sh, sem):
    c = lax.axis_index("c")
    s = lax.axis_index("s")
    wid = s * NC + c
    pltpu.sync_copy(zeros_hbm.at[pl.ds(s * RPS, RPS)],
                    acc_sh.at[pl.ds(s * RPS, RPS)])
    pltpu.sync_copy(src_hbm.at[wid], src_v)
    pltpu.sync_copy(dst_hbm.at[wid], dst_v)
    plsc.subcore_barrier()

    def body(j, carry):
        pltpu.async_copy(table_hbm.at[src_v.at[j]], rows_v, sem).wait()
        pltpu.sync_copy(rows_v, acc_sh.at[dst_v.at[j]], add=True)
        return carry

    lax.fori_loop(0, BPT, body, 0)
    plsc.subcore_barrier()
    pltpu.sync_copy(acc_sh.at[pl.ds(s * RPS, RPS)],
                    out_hbm.at[c, pl.ds(s * RPS, RPS)])


def _sc_hist(dst_r, zeros, ones):
    return pl.kernel(
        _sc_hist_body,
        out_type=jax.ShapeDtypeStruct((NC, RACC, D), jnp.float32),
        mesh=_mesh,
        scratch_types=[
            pltpu.VMEM((BPT, BLK), jnp.int32),
            pltpu.VMEM((BLK, D), jnp.float32),
            pltpu.VMEM_SHARED((RACC, D), jnp.float32),
            pltpu.SemaphoreType.DMA,
        ],
    )(dst_r, zeros, ones)


def _sc_scatter(table, src_r, dst_r, zeros):
    return pl.kernel(
        _sc_scatter_body,
        out_type=jax.ShapeDtypeStruct((NC, RACC, D), jnp.float32),
        mesh=_mesh,
        scratch_types=[
            pltpu.VMEM((BPT, BLK), jnp.int32),
            pltpu.VMEM((BPT, BLK), jnp.int32),
            pltpu.VMEM((BLK, D), jnp.float32),
            pltpu.VMEM_SHARED((RACC, D), jnp.float32),
            pltpu.SemaphoreType.DMA,
        ],
    )(table, src_r, dst_r, zeros)


RB = 400


def _mm_body(x_ref, w_ref, o_ref):
    o_ref[...] = jnp.dot(x_ref[...], w_ref[...],
                         preferred_element_type=jnp.float32)


def _tc_matmul(x, w):
    m, k = x.shape
    n = w.shape[1]
    return pl.pallas_call(
        _mm_body,
        grid=(m // RB,),
        in_specs=[pl.BlockSpec((RB, k), lambda i: (i, 0)),
                  pl.BlockSpec((k, n), lambda i: (0, 0))],
        out_specs=pl.BlockSpec((RB, n), lambda i: (i, 0)),
        out_shape=jax.ShapeDtypeStruct((m, n), jnp.float32),
    )(x, w)


def _scale_body(dp_ref, p_ref, h1s_ref, dis_ref):
    deg = dp_ref[0] + dp_ref[1] + 1.0
    dis = lax.rsqrt(deg)
    dis_ref[...] = dis
    h1s_ref[...] = p_ref[...] * dis


def _tc_scale(deg_part, p):
    return pl.pallas_call(
        _scale_body,
        grid=(N // RB,),
        in_specs=[pl.BlockSpec((NC, RB, 1), lambda i: (0, i, 0)),
                  pl.BlockSpec((RB, 128), lambda i: (i, 0))],
        out_specs=[pl.BlockSpec((RB, 128), lambda i: (i, 0)),
                   pl.BlockSpec((RB, 1), lambda i: (i, 0))],
        out_shape=[jax.ShapeDtypeStruct((N, 128), jnp.float32),
                   jax.ShapeDtypeStruct((N, 1), jnp.float32)],
    )(deg_part, p)


def _layer2_body(acc_ref, h1s_ref, dis_ref, b1_ref, w2_ref, h2s_ref):
    dis = dis_ref[...]
    out1 = (acc_ref[0] + acc_ref[1] + h1s_ref[...]) * dis + b1_ref[...]
    r = jnp.maximum(out1, 0.0)
    q = jnp.dot(r, w2_ref[...], preferred_element_type=jnp.float32)
    h2s_ref[...] = jnp.concatenate(
        [q * dis, jnp.zeros((RB, 64), jnp.float32)], axis=1)


def _tc_layer2(acc1, h1s, dis, b1, W2):
    return pl.pallas_call(
        _layer2_body,
        grid=(N // RB,),
        in_specs=[pl.BlockSpec((NC, RB, 128), lambda i: (0, i, 0)),
                  pl.BlockSpec((RB, 128), lambda i: (i, 0)),
                  pl.BlockSpec((RB, 1), lambda i: (i, 0)),
                  pl.BlockSpec((1, 128), lambda i: (0, 0)),
                  pl.BlockSpec((128, 64), lambda i: (0, 0))],
        out_specs=pl.BlockSpec((RB, 128), lambda i: (i, 0)),
        out_shape=jax.ShapeDtypeStruct((N, 128), jnp.float32),
    )(acc1, h1s, dis, b1.reshape(1, 128), W2)


def _final_body(acc_ref, h2s_ref, dis_ref, b2_ref, wd_ref, bd_ref,
                lp_ref, emb_ref):
    dis = dis_ref[...]
    out2 = (acc_ref[0] + acc_ref[1] + h2s_ref[...]) * dis + b2_ref[...]
    nrm = jnp.sqrt(jnp.sum(out2 * out2, axis=1, keepdims=True))
    emb = out2 / (nrm + 1e-12)
    emb_ref[...] = emb
    logits = jnp.dot(emb, wd_ref[...],
                     preferred_element_type=jnp.float32) + bd_ref[...]
    m = jnp.max(logits, axis=1, keepdims=True)
    lse = m + jnp.log(jnp.sum(jnp.exp(logits - m), axis=1, keepdims=True))
    lp_ref[...] = logits - lse


def _tc_final(acc2, h2s, dis, b2, Wd, bd):
    return pl.pallas_call(
        _final_body,
        grid=(N // RB,),
        in_specs=[pl.BlockSpec((NC, RB, 64), lambda i: (0, i, 0)),
                  pl.BlockSpec((RB, 64), lambda i: (i, 0)),
                  pl.BlockSpec((RB, 1), lambda i: (i, 0)),
                  pl.BlockSpec((1, 64), lambda i: (0, 0)),
                  pl.BlockSpec((64, 128), lambda i: (0, 0)),
                  pl.BlockSpec((1, 128), lambda i: (0, 0))],
        out_specs=[pl.BlockSpec((RB, 128), lambda i: (i, 0)),
                   pl.BlockSpec((RB, 64), lambda i: (i, 0))],
        out_shape=[jax.ShapeDtypeStruct((N, 128), jnp.float32),
                   jax.ShapeDtypeStruct((N, 64), jnp.float32)],
    )(acc2, h2s, dis, b2.reshape(1, 64), Wd, bd.reshape(1, 128))


def kernel(x, edge_index, W1, b1, W2, b2, Wd, bd):
    ei = edge_index.astype(jnp.int32)
    npad = EPAD - E
    pad_dst = N + (jnp.arange(npad, dtype=jnp.int32) % (RACC - N))
    src = jnp.concatenate([ei[0], jnp.zeros((npad,), jnp.int32)])
    dst = jnp.concatenate([ei[1], pad_dst])
    src_r = src.reshape(NW, BPT, BLK)
    dst_r = dst.reshape(NW, BPT, BLK)

    zeros = jnp.zeros((RACC, D), jnp.float32)
    ones = jnp.ones((BLK, D), jnp.float32)

    deg_part = _sc_hist(dst_r, zeros, ones)
    p = _tc_matmul(x, W1)
    h1s, dis = _tc_scale(deg_part[:, :N, 0:1], p)

    acc1 = _sc_scatter(h1s, src_r, dst_r, zeros)
    h2s = _tc_layer2(acc1[:, :N], h1s, dis, b1, W2)

    acc2 = _sc_scatter(h2s, src_r, dst_r, zeros)
    log_probs, emb = _tc_final(acc2[:, :N, :64], h2s[:, :64], dis, b2, Wd, bd)
    return (log_probs, emb)


# speedup vs baseline: 23.7484x; 1.9273x over previous
import jax
import jax.numpy as jnp
from jax import lax
from jax.experimental import pallas as pl
from jax.experimental.pallas import tpu as pltpu
from jax.experimental.pallas import tpu_sc as plsc

N = 10000
E = 320000
NC = 2
NS = 16
NW = NC * NS
BLK = 128
BPT = 80
EPAD = NW * BLK * BPT
SLOTS = 2
IS = 4
RACC = 10112
RPS = RACC // NS
D = 128

_mesh = plsc.VectorSubcoreMesh(core_axis_name="c", subcore_axis_name="s")


def _sc_hist_body(dst_hbm, zeros_hbm, ones_hbm, out_hbm, dst_v, ones_v,
                  acc_sh, sem):
    c = lax.axis_index("c")
    s = lax.axis_index("s")
    wid = s * NC + c
    pltpu.sync_copy(zeros_hbm.at[pl.ds(s * RPS, RPS)],
                    acc_sh.at[pl.ds(s * RPS, RPS)])
    pltpu.sync_copy(ones_hbm, ones_v)
    pltpu.sync_copy(dst_hbm.at[wid], dst_v)
    plsc.subcore_barrier()

    def body(j, carry):
        pltpu.sync_copy(ones_v, acc_sh.at[dst_v.at[j]], add=True)
        return carry

    lax.fori_loop(0, BPT, body, 0)
    plsc.subcore_barrier()
    pltpu.sync_copy(acc_sh.at[pl.ds(s * RPS, RPS)],
                    out_hbm.at[c, pl.ds(s * RPS, RPS)])


def _sc_scatter_body(table_hbm, src_hbm, dst_hbm, zeros_hbm, out_hbm,
                     srcix, dst_v, rows_v, acc_sh,
                     sg0, sg1, si0, si1, si2, si3):
    c = lax.axis_index("c")
    s = lax.axis_index("s")
    wid = s * NC + c
    sg = (sg0, sg1)
    si = (si0, si1, si2, si3)
    pltpu.sync_copy(zeros_hbm.at[pl.ds(s * RPS, RPS)],
                    acc_sh.at[pl.ds(s * RPS, RPS)])
    pltpu.sync_copy(dst_hbm.at[wid], dst_v)

    def fire_idx(j, islot):
        pltpu.async_copy(src_hbm.at[wid, j], srcix.at[islot], si[islot])

    def wait_idx(j, islot):
        pltpu.make_async_copy(src_hbm.at[wid, j], srcix.at[islot],
                              si[islot]).wait()

    def fire_g(j, islot, b):
        pltpu.async_copy(table_hbm.at[srcix.at[islot]], rows_v.at[b], sg[b])

    def wait_g(j, b):
        pltpu.make_async_copy(table_hbm.at[srcix.at[0]], rows_v.at[b],
                              sg[b]).wait()

    for k in range(IS):
        fire_idx(k, k)
    plsc.subcore_barrier()
    for k in range(SLOTS):
        wait_idx(k, k)
        fire_g(k, k, k)

    def step(i, carry):
        for u in range(IS):
            j = i * IS + u
            b = u % SLOTS
            wait_g(j, b)
            pltpu.sync_copy(rows_v.at[b], acc_sh.at[dst_v.at[j]], add=True)
            fire_idx(j + IS, u)
            wait_idx(j + SLOTS, (u + SLOTS) % IS)
            fire_g(j + SLOTS, (u + SLOTS) % IS, b)
        return carry

    lax.fori_loop(0, BPT // IS - 1, step, 0)
    for u in range(IS):
        j = BPT - IS + u
        b = u % SLOTS
        wait_g(j, b)
        pltpu.sync_copy(rows_v.at[b], acc_sh.at[dst_v.at[j]], add=True)
        if j + SLOTS < BPT:
            wait_idx(j + SLOTS, (u + SLOTS) % IS)
            fire_g(j + SLOTS, (u + SLOTS) % IS, b)
    plsc.subcore_barrier()
    pltpu.sync_copy(acc_sh.at[pl.ds(s * RPS, RPS)],
                    out_hbm.at[c, pl.ds(s * RPS, RPS)])


def _sc_hist(dst_r, zeros, ones):
    return pl.kernel(
        _sc_hist_body,
        out_type=jax.ShapeDtypeStruct((NC, RACC, D), jnp.float32),
        mesh=_mesh,
        scratch_types=[
            pltpu.VMEM((BPT, BLK), jnp.int32),
            pltpu.VMEM((BLK, D), jnp.float32),
            pltpu.VMEM_SHARED((RACC, D), jnp.float32),
            pltpu.SemaphoreType.DMA,
        ],
    )(dst_r, zeros, ones)


def _sc_scatter(table, src_r, dst_r, zeros):
    return pl.kernel(
        _sc_scatter_body,
        out_type=jax.ShapeDtypeStruct((NC, RACC, D), jnp.float32),
        mesh=_mesh,
        scratch_types=[
            pltpu.VMEM((IS, BLK), jnp.int32),
            pltpu.VMEM((BPT, BLK), jnp.int32),
            pltpu.VMEM((SLOTS, BLK, D), jnp.float32),
            pltpu.VMEM_SHARED((RACC, D), jnp.float32),
            pltpu.SemaphoreType.DMA,
            pltpu.SemaphoreType.DMA,
            pltpu.SemaphoreType.DMA,
            pltpu.SemaphoreType.DMA,
            pltpu.SemaphoreType.DMA,
            pltpu.SemaphoreType.DMA,
        ],
    )(table, src_r, dst_r, zeros)


RB = 400


def _mm_body(x_ref, w_ref, o_ref):
    o_ref[...] = jnp.dot(x_ref[...], w_ref[...],
                         preferred_element_type=jnp.float32)


def _tc_matmul(x, w):
    m, k = x.shape
    n = w.shape[1]
    return pl.pallas_call(
        _mm_body,
        grid=(m // RB,),
        in_specs=[pl.BlockSpec((RB, k), lambda i: (i, 0)),
                  pl.BlockSpec((k, n), lambda i: (0, 0))],
        out_specs=pl.BlockSpec((RB, n), lambda i: (i, 0)),
        out_shape=jax.ShapeDtypeStruct((m, n), jnp.float32),
    )(x, w)


def _scale_body(dp_ref, p_ref, h1s_ref, dis_ref):
    deg = dp_ref[0] + dp_ref[1] + 1.0
    dis = lax.rsqrt(deg)
    dis_ref[...] = dis
    h1s_ref[...] = p_ref[...] * dis


def _tc_scale(deg_part, p):
    return pl.pallas_call(
        _scale_body,
        grid=(N // RB,),
        in_specs=[pl.BlockSpec((NC, RB, 1), lambda i: (0, i, 0)),
                  pl.BlockSpec((RB, 128), lambda i: (i, 0))],
        out_specs=[pl.BlockSpec((RB, 128), lambda i: (i, 0)),
                   pl.BlockSpec((RB, 1), lambda i: (i, 0))],
        out_shape=[jax.ShapeDtypeStruct((N, 128), jnp.float32),
                   jax.ShapeDtypeStruct((N, 1), jnp.float32)],
    )(deg_part, p)


def _layer2_body(acc_ref, h1s_ref, dis_ref, b1_ref, w2_ref, h2s_ref):
    dis = dis_ref[...]
    out1 = (acc_ref[0] + acc_ref[1] + h1s_ref[...]) * dis + b1_ref[...]
    r = jnp.maximum(out1, 0.0)
    q = jnp.dot(r, w2_ref[...], preferred_element_type=jnp.float32)
    h2s_ref[...] = jnp.concatenate(
        [q * dis, jnp.zeros((RB, 64), jnp.float32)], axis=1)


def _tc_layer2(acc1, h1s, dis, b1, W2):
    return pl.pallas_call(
        _layer2_body,
        grid=(N // RB,),
        in_specs=[pl.BlockSpec((NC, RB, 128), lambda i: (0, i, 0)),
                  pl.BlockSpec((RB, 128), lambda i: (i, 0)),
                  pl.BlockSpec((RB, 1), lambda i: (i, 0)),
                  pl.BlockSpec((1, 128), lambda i: (0, 0)),
                  pl.BlockSpec((128, 64), lambda i: (0, 0))],
        out_specs=pl.BlockSpec((RB, 128), lambda i: (i, 0)),
        out_shape=jax.ShapeDtypeStruct((N, 128), jnp.float32),
    )(acc1, h1s, dis, b1.reshape(1, 128), W2)


def _final_body(acc_ref, h2s_ref, dis_ref, b2_ref, wd_ref, bd_ref,
                lp_ref, emb_ref):
    dis = dis_ref[...]
    out2 = (acc_ref[0] + acc_ref[1] + h2s_ref[...]) * dis + b2_ref[...]
    nrm = jnp.sqrt(jnp.sum(out2 * out2, axis=1, keepdims=True))
    emb = out2 / (nrm + 1e-12)
    emb_ref[...] = emb
    logits = jnp.dot(emb, wd_ref[...],
                     preferred_element_type=jnp.float32) + bd_ref[...]
    m = jnp.max(logits, axis=1, keepdims=True)
    lse = m + jnp.log(jnp.sum(jnp.exp(logits - m), axis=1, keepdims=True))
    lp_ref[...] = logits - lse


def _tc_final(acc2, h2s, dis, b2, Wd, bd):
    return pl.pallas_call(
        _final_body,
        grid=(N // RB,),
        in_specs=[pl.BlockSpec((NC, RB, 64), lambda i: (0, i, 0)),
                  pl.BlockSpec((RB, 64), lambda i: (i, 0)),
                  pl.BlockSpec((RB, 1), lambda i: (i, 0)),
                  pl.BlockSpec((1, 64), lambda i: (0, 0)),
                  pl.BlockSpec((64, 128), lambda i: (0, 0)),
                  pl.BlockSpec((1, 128), lambda i: (0, 0))],
        out_specs=[pl.BlockSpec((RB, 128), lambda i: (i, 0)),
                   pl.BlockSpec((RB, 64), lambda i: (i, 0))],
        out_shape=[jax.ShapeDtypeStruct((N, 128), jnp.float32),
                   jax.ShapeDtypeStruct((N, 64), jnp.float32)],
    )(acc2, h2s, dis, b2.reshape(1, 64), Wd, bd.reshape(1, 128))


def kernel(x, edge_index, W1, b1, W2, b2, Wd, bd):
    ei = edge_index.astype(jnp.int32)
    npad = EPAD - E
    pad_dst = N + (jnp.arange(npad, dtype=jnp.int32) % (RACC - N))
    pad_src = jnp.arange(npad, dtype=jnp.int32) % N
    src = jnp.concatenate([ei[0], pad_src])
    dst = jnp.concatenate([ei[1], pad_dst])
    src_r = src.reshape(NW, BPT, BLK)
    dst_r = dst.reshape(NW, BPT, BLK)

    zeros = jnp.zeros((RACC, D), jnp.float32)
    ones = jnp.ones((BLK, D), jnp.float32)

    deg_part = _sc_hist(dst_r, zeros, ones)
    p = _tc_matmul(x, W1)
    h1s, dis = _tc_scale(deg_part[:, :N, 0:1], p)

    acc1 = _sc_scatter(h1s, src_r, dst_r, zeros)
    h2s = _tc_layer2(acc1[:, :N], h1s, dis, b1, W2)

    acc2 = _sc_scatter(h2s, src_r, dst_r, zeros)
    log_probs, emb = _tc_final(acc2[:, :N, :64], h2s[:, :64], dis, b2, Wd, bd)
    return (log_probs, emb)


# speedup vs baseline: 24.6980x; 1.0400x over previous
import jax
import jax.numpy as jnp
from jax import lax
from jax.experimental import pallas as pl
from jax.experimental.pallas import tpu as pltpu
from jax.experimental.pallas import tpu_sc as plsc

N = 10000
E = 320000
NC = 2
NS = 16
NW = NC * NS
BLK = 128
BPT = 84
EPAD = NW * BLK * BPT
SLOTS = 3
IS = 4
RACC = 10112
RPS = RACC // NS
D = 128

_mesh = plsc.VectorSubcoreMesh(core_axis_name="c", subcore_axis_name="s")


def _sc_hist_body(dst_hbm, zeros_hbm, ones_hbm, out_hbm, dst_v, ones_v,
                  acc_sh, sem):
    c = lax.axis_index("c")
    s = lax.axis_index("s")
    wid = s * NC + c
    pltpu.sync_copy(zeros_hbm.at[pl.ds(s * RPS, RPS)],
                    acc_sh.at[pl.ds(s * RPS, RPS)])
    pltpu.sync_copy(ones_hbm, ones_v)
    pltpu.sync_copy(dst_hbm.at[wid], dst_v)
    plsc.subcore_barrier()

    def body(j, carry):
        pltpu.sync_copy(ones_v, acc_sh.at[dst_v.at[j]], add=True)
        return carry

    lax.fori_loop(0, BPT, body, 0)
    plsc.subcore_barrier()
    pltpu.sync_copy(acc_sh.at[pl.ds(s * RPS, RPS)],
                    out_hbm.at[c, pl.ds(s * RPS, RPS)])


def _sc_scatter_body(table_hbm, src_hbm, dst_hbm, zeros_hbm, out_hbm,
                     srcix, dstix, rows_v, acc_sh,
                     sg0, sg1, sg2, si0, si1, si2, si3, sd0, sd1, sd2, ss):
    c = lax.axis_index("c")
    s = lax.axis_index("s")
    wid = s * NC + c
    sg = (sg0, sg1, sg2)
    si = (si0, si1, si2, si3)
    sd = (sd0, sd1, sd2)
    pltpu.sync_copy(zeros_hbm.at[pl.ds(s * RPS, RPS)],
                    acc_sh.at[pl.ds(s * RPS, RPS)])

    def fire_si(j, k):
        pltpu.async_copy(src_hbm.at[wid, j], srcix.at[k], si[k])

    def wait_si(j, k):
        pltpu.make_async_copy(src_hbm.at[wid, j], srcix.at[k], si[k]).wait()

    def fire_di(j, k):
        pltpu.async_copy(dst_hbm.at[wid, j], dstix.at[k], sd[k])

    def wait_di(j, k):
        pltpu.make_async_copy(dst_hbm.at[wid, j], dstix.at[k], sd[k]).wait()

    def fire_g(k, b):
        pltpu.async_copy(table_hbm.at[srcix.at[k]], rows_v.at[b], sg[b])

    def wait_g(b):
        pltpu.make_async_copy(table_hbm.at[srcix.at[0]], rows_v.at[b],
                              sg[b]).wait()

    for k in range(IS):
        fire_si(k, k)
    for k in range(SLOTS):
        fire_di(k, k)
    plsc.subcore_barrier()
    for k in range(SLOTS):
        wait_si(k, k)
        fire_g(k, k)

    def block(j, b, ksi, fire_next_si, fire_next):
        wait_g(b)
        wait_di(j, b)
        h = pltpu.async_copy(rows_v.at[b], acc_sh.at[dstix.at[b]], ss,
                             add=True)
        if fire_next_si:
            fire_si(j + IS, ksi)
        h.wait()
        if fire_next:
            fire_di(j + SLOTS, b)
            wait_si(j + SLOTS, (ksi + SLOTS) % IS)
            fire_g((ksi + SLOTS) % IS, b)

    def step(i, carry):
        for u in range(12):
            j = i * 12 + u
            block(j, u % SLOTS, u % IS, True, True)
        return carry

    lax.fori_loop(0, BPT // 12 - 1, step, 0)
    for u in range(12):
        j = BPT - 12 + u
        block(j, u % SLOTS, u % IS, j + IS < BPT, j + SLOTS < BPT)
    plsc.subcore_barrier()
    pltpu.sync_copy(acc_sh.at[pl.ds(s * RPS, RPS)],
                    out_hbm.at[c, pl.ds(s * RPS, RPS)])


def _sc_hist(dst_r, zeros, ones):
    return pl.kernel(
        _sc_hist_body,
        out_type=jax.ShapeDtypeStruct((NC, RACC, D), jnp.float32),
        mesh=_mesh,
        scratch_types=[
            pltpu.VMEM((BPT, BLK), jnp.int32),
            pltpu.VMEM((BLK, D), jnp.float32),
            pltpu.VMEM_SHARED((RACC, D), jnp.float32),
            pltpu.SemaphoreType.DMA,
        ],
    )(dst_r, zeros, ones)


def _sc_scatter(table, src_r, dst_r, zeros):
    return pl.kernel(
        _sc_scatter_body,
        out_type=jax.ShapeDtypeStruct((NC, RACC, D), jnp.float32),
        mesh=_mesh,
        scratch_types=[
            pltpu.VMEM((IS, BLK), jnp.int32),
            pltpu.VMEM((SLOTS, BLK), jnp.int32),
            pltpu.VMEM((SLOTS, BLK, D), jnp.float32),
            pltpu.VMEM_SHARED((RACC, D), jnp.float32),
        ] + [pltpu.SemaphoreType.DMA] * 11,
    )(table, src_r, dst_r, zeros)


RB = 400


def _mm_body(x_ref, w_ref, o_ref):
    o_ref[...] = jnp.dot(x_ref[...], w_ref[...],
                         preferred_element_type=jnp.float32)


def _tc_matmul(x, w):
    m, k = x.shape
    n = w.shape[1]
    return pl.pallas_call(
        _mm_body,
        grid=(m // RB,),
        in_specs=[pl.BlockSpec((RB, k), lambda i: (i, 0)),
                  pl.BlockSpec((k, n), lambda i: (0, 0))],
        out_specs=pl.BlockSpec((RB, n), lambda i: (i, 0)),
        out_shape=jax.ShapeDtypeStruct((m, n), jnp.float32),
    )(x, w)


def _scale_body(dp_ref, p_ref, h1s_ref, dis_ref):
    deg = dp_ref[0] + dp_ref[1] + 1.0
    dis = lax.rsqrt(deg)
    dis_ref[...] = dis
    h1s_ref[...] = p_ref[...] * dis


def _tc_scale(deg_part, p):
    return pl.pallas_call(
        _scale_body,
        grid=(N // RB,),
        in_specs=[pl.BlockSpec((NC, RB, 1), lambda i: (0, i, 0)),
                  pl.BlockSpec((RB, 128), lambda i: (i, 0))],
        out_specs=[pl.BlockSpec((RB, 128), lambda i: (i, 0)),
                   pl.BlockSpec((RB, 1), lambda i: (i, 0))],
        out_shape=[jax.ShapeDtypeStruct((N, 128), jnp.float32),
                   jax.ShapeDtypeStruct((N, 1), jnp.float32)],
    )(deg_part, p)


def _layer2_body(acc_ref, h1s_ref, dis_ref, b1_ref, w2_ref, h2s_ref):
    dis = dis_ref[...]
    out1 = (acc_ref[0] + acc_ref[1] + h1s_ref[...]) * dis + b1_ref[...]
    r = jnp.maximum(out1, 0.0)
    q = jnp.dot(r, w2_ref[...], preferred_element_type=jnp.float32)
    h2s_ref[...] = jnp.concatenate(
        [q * dis, jnp.zeros((RB, 64), jnp.float32)], axis=1)


def _tc_layer2(acc1, h1s, dis, b1, W2):
    return pl.pallas_call(
        _layer2_body,
        grid=(N // RB,),
        in_specs=[pl.BlockSpec((NC, RB, 128), lambda i: (0, i, 0)),
                  pl.BlockSpec((RB, 128), lambda i: (i, 0)),
                  pl.BlockSpec((RB, 1), lambda i: (i, 0)),
                  pl.BlockSpec((1, 128), lambda i: (0, 0)),
                  pl.BlockSpec((128, 64), lambda i: (0, 0))],
        out_specs=pl.BlockSpec((RB, 128), lambda i: (i, 0)),
        out_shape=jax.ShapeDtypeStruct((N, 128), jnp.float32),
    )(acc1, h1s, dis, b1.reshape(1, 128), W2)


def _final_body(acc_ref, h2s_ref, dis_ref, b2_ref, wd_ref, bd_ref,
                lp_ref, emb_ref):
    dis = dis_ref[...]
    out2 = (acc_ref[0] + acc_ref[1] + h2s_ref[...]) * dis + b2_ref[...]
    nrm = jnp.sqrt(jnp.sum(out2 * out2, axis=1, keepdims=True))
    emb = out2 / (nrm + 1e-12)
    emb_ref[...] = emb
    logits = jnp.dot(emb, wd_ref[...],
                     preferred_element_type=jnp.float32) + bd_ref[...]
    m = jnp.max(logits, axis=1, keepdims=True)
    lse = m + jnp.log(jnp.sum(jnp.exp(logits - m), axis=1, keepdims=True))
    lp_ref[...] = logits - lse


def _tc_final(acc2, h2s, dis, b2, Wd, bd):
    return pl.pallas_call(
        _final_body,
        grid=(N // RB,),
        in_specs=[pl.BlockSpec((NC, RB, 64), lambda i: (0, i, 0)),
                  pl.BlockSpec((RB, 64), lambda i: (i, 0)),
                  pl.BlockSpec((RB, 1), lambda i: (i, 0)),
                  pl.BlockSpec((1, 64), lambda i: (0, 0)),
                  pl.BlockSpec((64, 128), lambda i: (0, 0)),
                  pl.BlockSpec((1, 128), lambda i: (0, 0))],
        out_specs=[pl.BlockSpec((RB, 128), lambda i: (i, 0)),
                   pl.BlockSpec((RB, 64), lambda i: (i, 0))],
        out_shape=[jax.ShapeDtypeStruct((N, 128), jnp.float32),
                   jax.ShapeDtypeStruct((N, 64), jnp.float32)],
    )(acc2, h2s, dis, b2.reshape(1, 64), Wd, bd.reshape(1, 128))


def kernel(x, edge_index, W1, b1, W2, b2, Wd, bd):
    ei = edge_index.astype(jnp.int32)
    npad = EPAD - E
    pad_dst = N + (jnp.arange(npad, dtype=jnp.int32) % (RACC - N))
    pad_src = jnp.arange(npad, dtype=jnp.int32) % N
    src = jnp.concatenate([ei[0], pad_src])
    dst = jnp.concatenate([ei[1], pad_dst])
    src_r = src.reshape(NW, BPT, BLK)
    dst_r = dst.reshape(NW, BPT, BLK)

    zeros = jnp.zeros((RACC, D), jnp.float32)
    ones = jnp.ones((BLK, D), jnp.float32)

    deg_part = _sc_hist(dst_r, zeros, ones)
    p = _tc_matmul(x, W1)
    h1s, dis = _tc_scale(deg_part[:, :N, 0:1], p)

    acc1 = _sc_scatter(h1s, src_r, dst_r, zeros)
    h2s = _tc_layer2(acc1[:, :N], h1s, dis, b1, W2)

    acc2 = _sc_scatter(h2s, src_r, dst_r, zeros)
    log_probs, emb = _tc_final(acc2[:, :N, :64], h2s[:, :64], dis, b2, Wd, bd)
    return (log_probs, emb)


# speedup vs baseline: 24.7025x; 1.0002x over previous
import jax
import jax.numpy as jnp
from jax import lax
from jax.experimental import pallas as pl
from jax.experimental.pallas import tpu as pltpu
from jax.experimental.pallas import tpu_sc as plsc

N = 10000
E = 320000
NC = 2
NS = 16
NW = NC * NS
BLK = 128
BPT = 84
EPAD = NW * BLK * BPT
SLOTS = 3
IS = 4
RACC = 10112
RPS = RACC // NS
D = 128

_mesh = plsc.VectorSubcoreMesh(core_axis_name="c", subcore_axis_name="s")


def _sc_hist_body(dst_hbm, zeros_hbm, ones_hbm, out_hbm, dst_v, ones_v,
                  acc_sh, sem):
    c = lax.axis_index("c")
    s = lax.axis_index("s")
    wid = s * NC + c
    pltpu.sync_copy(zeros_hbm.at[pl.ds(s * RPS, RPS)],
                    acc_sh.at[pl.ds(s * RPS, RPS)])
    pltpu.sync_copy(ones_hbm, ones_v)
    pltpu.sync_copy(dst_hbm.at[wid], dst_v)
    plsc.subcore_barrier()

    def fire(j):
        pltpu.async_copy(ones_v, acc_sh.at[dst_v.at[j]], sem, add=True)

    def drain(j):
        pltpu.make_async_copy(ones_v, acc_sh.at[dst_v.at[j]], sem).wait()

    for k in range(4):
        fire(k)

    def body(j, carry):
        drain(j)
        fire(j + 4)
        return carry

    lax.fori_loop(0, BPT - 4, body, 0)
    for k in range(BPT - 4, BPT):
        drain(k)
    plsc.subcore_barrier()
    pltpu.sync_copy(acc_sh.at[pl.ds(s * RPS, RPS)],
                    out_hbm.at[c, pl.ds(s * RPS, RPS)])


def _sc_scatter_body(table_hbm, src_hbm, dst_hbm, zeros_hbm, out_hbm,
                     srcix, dstix, rows_v, acc_sh,
                     sg0, sg1, sg2, si0, si1, si2, si3, sd0, sd1, sd2, ss):
    c = lax.axis_index("c")
    s = lax.axis_index("s")
    wid = s * NC + c
    sg = (sg0, sg1, sg2)
    si = (si0, si1, si2, si3)
    sd = (sd0, sd1, sd2)
    pltpu.sync_copy(zeros_hbm.at[pl.ds(s * RPS, RPS)],
                    acc_sh.at[pl.ds(s * RPS, RPS)])

    def fire_si(j, k):
        pltpu.async_copy(src_hbm.at[wid, j], srcix.at[k], si[k])

    def wait_si(j, k):
        pltpu.make_async_copy(src_hbm.at[wid, j], srcix.at[k], si[k]).wait()

    def fire_di(j, k):
        pltpu.async_copy(dst_hbm.at[wid, j], dstix.at[k], sd[k])

    def wait_di(j, k):
        pltpu.make_async_copy(dst_hbm.at[wid, j], dstix.at[k], sd[k]).wait()

    def fire_g(k, b):
        pltpu.async_copy(table_hbm.at[srcix.at[k]], rows_v.at[b], sg[b])

    def wait_g(b):
        pltpu.make_async_copy(table_hbm.at[srcix.at[0]], rows_v.at[b],
                              sg[b]).wait()

    for k in range(IS):
        fire_si(k, k)
    for k in range(SLOTS):
        fire_di(k, k)
    plsc.subcore_barrier()
    for k in range(SLOTS):
        wait_si(k, k)
        fire_g(k, k)

    def block(j, b, ksi, fire_next_si, fire_next):
        wait_g(b)
        wait_di(j, b)
        h = pltpu.async_copy(rows_v.at[b], acc_sh.at[dstix.at[b]], ss,
                             add=True)
        if fire_next_si:
            fire_si(j + IS, ksi)
        h.wait()
        if fire_next:
            fire_di(j + SLOTS, b)
            wait_si(j + SLOTS, (ksi + SLOTS) % IS)
            fire_g((ksi + SLOTS) % IS, b)

    def step(i, carry):
        for u in range(12):
            j = i * 12 + u
            block(j, u % SLOTS, u % IS, True, True)
        return carry

    lax.fori_loop(0, BPT // 12 - 1, step, 0)
    for u in range(12):
        j = BPT - 12 + u
        block(j, u % SLOTS, u % IS, j + IS < BPT, j + SLOTS < BPT)
    plsc.subcore_barrier()
    pltpu.sync_copy(acc_sh.at[pl.ds(s * RPS, RPS)],
                    out_hbm.at[c, pl.ds(s * RPS, RPS)])


def _sc_hist(dst_r, zeros, ones):
    return pl.kernel(
        _sc_hist_body,
        out_type=jax.ShapeDtypeStruct((NC, RACC, D), jnp.float32),
        mesh=_mesh,
        scratch_types=[
            pltpu.VMEM((BPT, BLK), jnp.int32),
            pltpu.VMEM((BLK, D), jnp.float32),
            pltpu.VMEM_SHARED((RACC, D), jnp.float32),
            pltpu.SemaphoreType.DMA,
        ],
    )(dst_r, zeros, ones)


def _sc_scatter(table, src_r, dst_r, zeros):
    return pl.kernel(
        _sc_scatter_body,
        out_type=jax.ShapeDtypeStruct((NC, RACC, D), jnp.float32),
        mesh=_mesh,
        scratch_types=[
            pltpu.VMEM((IS, BLK), jnp.int32),
            pltpu.VMEM((SLOTS, BLK), jnp.int32),
            pltpu.VMEM((SLOTS, BLK, D), jnp.float32),
            pltpu.VMEM_SHARED((RACC, D), jnp.float32),
        ] + [pltpu.SemaphoreType.DMA] * 11,
    )(table, src_r, dst_r, zeros)


RB = 400


def _mm_body(x_ref, w_ref, o_ref):
    o_ref[...] = jnp.dot(x_ref[...], w_ref[...],
                         preferred_element_type=jnp.float32)


def _tc_matmul(x, w):
    m, k = x.shape
    n = w.shape[1]
    return pl.pallas_call(
        _mm_body,
        grid=(m // RB,),
        in_specs=[pl.BlockSpec((RB, k), lambda i: (i, 0)),
                  pl.BlockSpec((k, n), lambda i: (0, 0))],
        out_specs=pl.BlockSpec((RB, n), lambda i: (i, 0)),
        out_shape=jax.ShapeDtypeStruct((m, n), jnp.float32),
    )(x, w)


def _scale_body(dp_ref, p_ref, h1s_ref, dis_ref):
    deg = dp_ref[0] + dp_ref[1] + 1.0
    dis = lax.rsqrt(deg)
    dis_ref[...] = dis
    h1s_ref[...] = p_ref[...] * dis


def _tc_scale(deg_part, p):
    return pl.pallas_call(
        _scale_body,
        grid=(N // RB,),
        in_specs=[pl.BlockSpec((NC, RB, 1), lambda i: (0, i, 0)),
                  pl.BlockSpec((RB, 128), lambda i: (i, 0))],
        out_specs=[pl.BlockSpec((RB, 128), lambda i: (i, 0)),
                   pl.BlockSpec((RB, 1), lambda i: (i, 0))],
        out_shape=[jax.ShapeDtypeStruct((N, 128), jnp.float32),
                   jax.ShapeDtypeStruct((N, 1), jnp.float32)],
    )(deg_part, p)


def _layer2_body(acc_ref, h1s_ref, dis_ref, b1_ref, w2_ref, h2s_ref):
    dis = dis_ref[...]
    out1 = (acc_ref[0] + acc_ref[1] + h1s_ref[...]) * dis + b1_ref[...]
    r = jnp.maximum(out1, 0.0)
    q = jnp.dot(r, w2_ref[...], preferred_element_type=jnp.float32)
    h2s_ref[...] = jnp.concatenate(
        [q * dis, jnp.zeros((RB, 64), jnp.float32)], axis=1)


def _tc_layer2(acc1, h1s, dis, b1, W2):
    return pl.pallas_call(
        _layer2_body,
        grid=(N // RB,),
        in_specs=[pl.BlockSpec((NC, RB, 128), lambda i: (0, i, 0)),
                  pl.BlockSpec((RB, 128), lambda i: (i, 0)),
                  pl.BlockSpec((RB, 1), lambda i: (i, 0)),
                  pl.BlockSpec((1, 128), lambda i: (0, 0)),
                  pl.BlockSpec((128, 64), lambda i: (0, 0))],
        out_specs=pl.BlockSpec((RB, 128), lambda i: (i, 0)),
        out_shape=jax.ShapeDtypeStruct((N, 128), jnp.float32),
    )(acc1, h1s, dis, b1.reshape(1, 128), W2)


def _final_body(acc_ref, h2s_ref, dis_ref, b2_ref, wd_ref, bd_ref,
                lp_ref, emb_ref):
    dis = dis_ref[...]
    out2 = (acc_ref[0] + acc_ref[1] + h2s_ref[...]) * dis + b2_ref[...]
    nrm = jnp.sqrt(jnp.sum(out2 * out2, axis=1, keepdims=True))
    emb = out2 / (nrm + 1e-12)
    emb_ref[...] = emb
    logits = jnp.dot(emb, wd_ref[...],
                     preferred_element_type=jnp.float32) + bd_ref[...]
    m = jnp.max(logits, axis=1, keepdims=True)
    lse = m + jnp.log(jnp.sum(jnp.exp(logits - m), axis=1, keepdims=True))
    lp_ref[...] = logits - lse


def _tc_final(acc2, h2s, dis, b2, Wd, bd):
    return pl.pallas_call(
        _final_body,
        grid=(N // RB,),
        in_specs=[pl.BlockSpec((NC, RB, 64), lambda i: (0, i, 0)),
                  pl.BlockSpec((RB, 64), lambda i: (i, 0)),
                  pl.BlockSpec((RB, 1), lambda i: (i, 0)),
                  pl.BlockSpec((1, 64), lambda i: (0, 0)),
                  pl.BlockSpec((64, 128), lambda i: (0, 0)),
                  pl.BlockSpec((1, 128), lambda i: (0, 0))],
        out_specs=[pl.BlockSpec((RB, 128), lambda i: (i, 0)),
                   pl.BlockSpec((RB, 64), lambda i: (i, 0))],
        out_shape=[jax.ShapeDtypeStruct((N, 128), jnp.float32),
                   jax.ShapeDtypeStruct((N, 64), jnp.float32)],
    )(acc2, h2s, dis, b2.reshape(1, 64), Wd, bd.reshape(1, 128))


def kernel(x, edge_index, W1, b1, W2, b2, Wd, bd):
    ei = edge_index.astype(jnp.int32)
    npad = EPAD - E
    pad_dst = N + (jnp.arange(npad, dtype=jnp.int32) % (RACC - N))
    pad_src = jnp.arange(npad, dtype=jnp.int32) % N
    src = jnp.concatenate([ei[0], pad_src])
    dst = jnp.concatenate([ei[1], pad_dst])
    src_r = src.reshape(NW, BPT, BLK)
    dst_r = dst.reshape(NW, BPT, BLK)

    zeros = jnp.zeros((RACC, D), jnp.float32)
    ones = jnp.ones((BLK, D), jnp.float32)

    deg_part = _sc_hist(dst_r, zeros, ones)
    p = _tc_matmul(x, W1)
    h1s, dis = _tc_scale(deg_part[:, :N, 0:1], p)

    acc1 = _sc_scatter(h1s, src_r, dst_r, zeros)
    h2s = _tc_layer2(acc1[:, :N], h1s, dis, b1, W2)

    acc2 = _sc_scatter(h2s, src_r, dst_r, zeros)
    log_probs, emb = _tc_final(acc2[:, :N, :64], h2s[:, :64], dis, b2, Wd, bd)
    return (log_probs, emb)


# speedup vs baseline: 27.8933x; 1.1292x over previous
import jax
import jax.numpy as jnp
from jax import lax
from jax.experimental import pallas as pl
from jax.experimental.pallas import tpu as pltpu
from jax.experimental.pallas import tpu_sc as plsc

N = 10000
E = 320000
NC = 2
NS = 16
NW = NC * NS
BLK = 128
BPT = 84
EPAD = NW * BLK * BPT
SLOTS = 3
IS = 4
RACC = 10112
RPS = RACC // NS
D = 128

_mesh = plsc.VectorSubcoreMesh(core_axis_name="c", subcore_axis_name="s")


def _sc_hist_body(dst_hbm, zeros_hbm, ones_hbm, out_hbm, dst_v, ones_v,
                  acc_sh, sem):
    c = lax.axis_index("c")
    s = lax.axis_index("s")
    wid = s * NC + c
    pltpu.sync_copy(zeros_hbm.at[pl.ds(s * RPS, RPS)],
                    acc_sh.at[pl.ds(s * RPS, RPS)])
    pltpu.sync_copy(ones_hbm, ones_v)
    pltpu.sync_copy(dst_hbm.at[wid], dst_v)
    plsc.subcore_barrier()

    def fire(j):
        pltpu.async_copy(ones_v, acc_sh.at[dst_v.at[j]], sem, add=True)

    def drain(j):
        pltpu.make_async_copy(ones_v, acc_sh.at[dst_v.at[j]], sem).wait()

    for k in range(4):
        fire(k)

    def body(j, carry):
        drain(j)
        fire(j + 4)
        return carry

    lax.fori_loop(0, BPT - 4, body, 0)
    for k in range(BPT - 4, BPT):
        drain(k)
    plsc.subcore_barrier()
    pltpu.sync_copy(acc_sh.at[pl.ds(s * RPS, RPS)],
                    out_hbm.at[c, pl.ds(s * RPS, RPS)])


def _sc_scatter_body(table_hbm, src_hbm, dst_hbm, zeros_hbm, out_hbm,
                     srcix, dstix, rows_v, acc_sh,
                     sg0, sg1, sg2, si0, si1, si2, si3, sd0, sd1, sd2, ss):
    c = lax.axis_index("c")
    s = lax.axis_index("s")
    wid = s * NC + c
    sg = (sg0, sg1, sg2)
    si = (si0, si1, si2, si3)
    sd = (sd0, sd1, sd2)
    pltpu.sync_copy(zeros_hbm.at[pl.ds(s * RPS, RPS)],
                    acc_sh.at[pl.ds(s * RPS, RPS)])

    def fire_si(j, k):
        pltpu.async_copy(src_hbm.at[wid, j], srcix.at[k], si[k])

    def wait_si(j, k):
        pltpu.make_async_copy(src_hbm.at[wid, j], srcix.at[k], si[k]).wait()

    def fire_di(j, k):
        pltpu.async_copy(dst_hbm.at[wid, j], dstix.at[k], sd[k])

    def wait_di(j, k):
        pltpu.make_async_copy(dst_hbm.at[wid, j], dstix.at[k], sd[k]).wait()

    def fire_g(k, b):
        pltpu.async_copy(table_hbm.at[srcix.at[k]], rows_v.at[b], sg[b])

    def wait_g(b):
        pltpu.make_async_copy(table_hbm.at[srcix.at[0]], rows_v.at[b],
                              sg[b]).wait()

    for k in range(IS):
        fire_si(k, k)
    for k in range(SLOTS):
        fire_di(k, k)
    plsc.subcore_barrier()
    for k in range(SLOTS):
        wait_si(k, k)
        fire_g(k, k)

    def block(j, b, ksi, fire_next_si, fire_next):
        wait_g(b)
        wait_di(j, b)
        h = pltpu.async_copy(rows_v.at[b], acc_sh.at[dstix.at[b]], ss,
                             add=True)
        if fire_next_si:
            fire_si(j + IS, ksi)
        h.wait()
        if fire_next:
            fire_di(j + SLOTS, b)
            wait_si(j + SLOTS, (ksi + SLOTS) % IS)
            fire_g((ksi + SLOTS) % IS, b)

    def step(i, carry):
        for u in range(12):
            j = i * 12 + u
            block(j, u % SLOTS, u % IS, True, True)
        return carry

    lax.fori_loop(0, BPT // 12 - 1, step, 0)
    for u in range(12):
        j = BPT - 12 + u
        block(j, u % SLOTS, u % IS, j + IS < BPT, j + SLOTS < BPT)
    plsc.subcore_barrier()
    pltpu.sync_copy(acc_sh.at[pl.ds(s * RPS, RPS)],
                    out_hbm.at[c, pl.ds(s * RPS, RPS)])


HW = 16


def _sc_hist(dst_r, zeros, ones):
    return pl.kernel(
        _sc_hist_body,
        out_type=jax.ShapeDtypeStruct((NC, RACC, HW), jnp.float32),
        mesh=_mesh,
        compiler_params=pltpu.CompilerParams(use_tc_tiling_on_sc=False),
        scratch_types=[
            pltpu.VMEM((BPT, BLK), jnp.int32),
            pltpu.VMEM((BLK, HW), jnp.float32),
            pltpu.VMEM_SHARED((RACC, HW), jnp.float32),
            pltpu.SemaphoreType.DMA,
        ],
    )(dst_r, zeros, ones)


def _sc_scatter(table, src_r, dst_r, zeros):
    return pl.kernel(
        _sc_scatter_body,
        out_type=jax.ShapeDtypeStruct((NC, RACC, D), jnp.float32),
        mesh=_mesh,
        scratch_types=[
            pltpu.VMEM((IS, BLK), jnp.int32),
            pltpu.VMEM((SLOTS, BLK), jnp.int32),
            pltpu.VMEM((SLOTS, BLK, D), jnp.float32),
            pltpu.VMEM_SHARED((RACC, D), jnp.float32),
        ] + [pltpu.SemaphoreType.DMA] * 11,
    )(table, src_r, dst_r, zeros)


RB = 400


def _mm_body(x_ref, w_ref, o_ref):
    o_ref[...] = jnp.dot(x_ref[...], w_ref[...],
                         preferred_element_type=jnp.float32)


def _tc_matmul(x, w):
    m, k = x.shape
    n = w.shape[1]
    return pl.pallas_call(
        _mm_body,
        grid=(m // RB,),
        in_specs=[pl.BlockSpec((RB, k), lambda i: (i, 0)),
                  pl.BlockSpec((k, n), lambda i: (0, 0))],
        out_specs=pl.BlockSpec((RB, n), lambda i: (i, 0)),
        out_shape=jax.ShapeDtypeStruct((m, n), jnp.float32),
    )(x, w)


def _scale_body(dp_ref, p_ref, h1s_ref, dis_ref):
    deg = dp_ref[0] + dp_ref[1] + 1.0
    dis = lax.rsqrt(deg)
    dis_ref[...] = dis
    h1s_ref[...] = p_ref[...] * dis


def _tc_scale(deg_part, p):
    return pl.pallas_call(
        _scale_body,
        grid=(N // RB,),
        in_specs=[pl.BlockSpec((NC, RB, 1), lambda i: (0, i, 0)),
                  pl.BlockSpec((RB, 128), lambda i: (i, 0))],
        out_specs=[pl.BlockSpec((RB, 128), lambda i: (i, 0)),
                   pl.BlockSpec((RB, 1), lambda i: (i, 0))],
        out_shape=[jax.ShapeDtypeStruct((N, 128), jnp.float32),
                   jax.ShapeDtypeStruct((N, 1), jnp.float32)],
    )(deg_part, p)


def _layer2_body(acc_ref, h1s_ref, dis_ref, b1_ref, w2_ref, h2s_ref):
    dis = dis_ref[...]
    out1 = (acc_ref[0] + acc_ref[1] + h1s_ref[...]) * dis + b1_ref[...]
    r = jnp.maximum(out1, 0.0)
    q = jnp.dot(r, w2_ref[...], preferred_element_type=jnp.float32)
    h2s_ref[...] = jnp.concatenate(
        [q * dis, jnp.zeros((RB, 64), jnp.float32)], axis=1)


def _tc_layer2(acc1, h1s, dis, b1, W2):
    return pl.pallas_call(
        _layer2_body,
        grid=(N // RB,),
        in_specs=[pl.BlockSpec((NC, RB, 128), lambda i: (0, i, 0)),
                  pl.BlockSpec((RB, 128), lambda i: (i, 0)),
                  pl.BlockSpec((RB, 1), lambda i: (i, 0)),
                  pl.BlockSpec((1, 128), lambda i: (0, 0)),
                  pl.BlockSpec((128, 64), lambda i: (0, 0))],
        out_specs=pl.BlockSpec((RB, 128), lambda i: (i, 0)),
        out_shape=jax.ShapeDtypeStruct((N, 128), jnp.float32),
    )(acc1, h1s, dis, b1.reshape(1, 128), W2)


def _final_body(acc_ref, h2s_ref, dis_ref, b2_ref, wd_ref, bd_ref,
                lp_ref, emb_ref):
    dis = dis_ref[...]
    out2 = (acc_ref[0] + acc_ref[1] + h2s_ref[...]) * dis + b2_ref[...]
    nrm = jnp.sqrt(jnp.sum(out2 * out2, axis=1, keepdims=True))
    emb = out2 / (nrm + 1e-12)
    emb_ref[...] = emb
    logits = jnp.dot(emb, wd_ref[...],
                     preferred_element_type=jnp.float32) + bd_ref[...]
    m = jnp.max(logits, axis=1, keepdims=True)
    lse = m + jnp.log(jnp.sum(jnp.exp(logits - m), axis=1, keepdims=True))
    lp_ref[...] = logits - lse


def _tc_final(acc2, h2s, dis, b2, Wd, bd):
    return pl.pallas_call(
        _final_body,
        grid=(N // RB,),
        in_specs=[pl.BlockSpec((NC, RB, 64), lambda i: (0, i, 0)),
                  pl.BlockSpec((RB, 64), lambda i: (i, 0)),
                  pl.BlockSpec((RB, 1), lambda i: (i, 0)),
                  pl.BlockSpec((1, 64), lambda i: (0, 0)),
                  pl.BlockSpec((64, 128), lambda i: (0, 0)),
                  pl.BlockSpec((1, 128), lambda i: (0, 0))],
        out_specs=[pl.BlockSpec((RB, 128), lambda i: (i, 0)),
                   pl.BlockSpec((RB, 64), lambda i: (i, 0))],
        out_shape=[jax.ShapeDtypeStruct((N, 128), jnp.float32),
                   jax.ShapeDtypeStruct((N, 64), jnp.float32)],
    )(acc2, h2s, dis, b2.reshape(1, 64), Wd, bd.reshape(1, 128))


def kernel(x, edge_index, W1, b1, W2, b2, Wd, bd):
    ei = edge_index.astype(jnp.int32)
    npad = EPAD - E
    pad_dst = N + (jnp.arange(npad, dtype=jnp.int32) % (RACC - N))
    pad_src = jnp.arange(npad, dtype=jnp.int32) % N
    src = jnp.concatenate([ei[0], pad_src])
    dst = jnp.concatenate([ei[1], pad_dst])
    src_r = src.reshape(NW, BPT, BLK)
    dst_r = dst.reshape(NW, BPT, BLK)

    zeros = jnp.zeros((RACC, D), jnp.float32)
    ones = jnp.ones((BLK, HW), jnp.float32)

    deg_part = _sc_hist(dst_r, zeros[:, :HW], ones)
    p = _tc_matmul(x, W1)
    h1s, dis = _tc_scale(deg_part[:, :N, 0:1], p)

    acc1 = _sc_scatter(h1s, src_r, dst_r, zeros)
    h2s = _tc_layer2(acc1[:, :N], h1s, dis, b1, W2)

    acc2 = _sc_scatter(h2s, src_r, dst_r, zeros)
    log_probs, emb = _tc_final(acc2[:, :N, :64], h2s[:, :64], dis, b2, Wd, bd)
    return (log_probs, emb)
